# baseline (speedup 1.0000x reference)
import math

import jax
import jax.numpy as jnp
from jax import lax
from jax.experimental import pallas as pl
from jax.experimental.pallas import tpu as pltpu

SQRT_2PI = math.sqrt(2.0 * math.pi)
NWALKERS = 65536
NELEC = 4
NDIM = 3
NSTEP = 16
D = NELEC * NDIM
WR, WC = 512, 128
LR = 0.2
EPS = 1e-7


def _tc_body(mean_ref, sigma_ref, perm_ref, z_ref, u_ref, out_ref,
             ms_s, pos_s):
    k = pl.program_id(0)

    @pl.when(k == 0)
    def _init():
        ms_s[0:1, :] = jnp.concatenate(
            [mean_ref[...], jnp.zeros((1, 128 - NDIM), jnp.float32)], axis=1)
        ms_s[1:2, :] = jnp.concatenate(
            [sigma_ref[...], jnp.zeros((1, 128 - NDIM), jnp.float32)], axis=1)

    def mdim(i):
        return ms_s[0:1, i:i + 1]

    def sdim(i):
        return ms_s[1:2, i:i + 1]

    j3 = lax.broadcasted_iota(jnp.int32, (D, 1, 1), 0) % NDIM
    mrow = jnp.where(j3 == 0, mdim(0), jnp.where(j3 == 1, mdim(1), mdim(2)))
    srow = jnp.where(j3 == 0, sdim(0), jnp.where(j3 == 1, sdim(1), sdim(2)))
    z = z_ref[0]
    x = mrow + srow * z

    @pl.when(k == 0)
    def _init_pos():
        pos_s[...] = x

    @pl.when(k > 0)
    def _step():
        p = jnp.exp(-(x * x) / 2.0) / SQRT_2PI
        f = p[0]
        for j in range(1, D):
            f = f * p[j]

        g = None
        for i in range(NDIM):
            t = x[i] - mdim(i)
            gt = jnp.exp(-(t * t) / (2.0 * sdim(i) * sdim(i))) / (
                SQRT_2PI * sdim(i))
            g = gt if g is None else g * gt

        mmax = jnp.max(f / g)
        u = u_ref[0]
        accepted = (g * u) * mmax < f
        pos_s[...] = jnp.where(accepted[None], x, pos_s[...])

        coef = (2.0 / NWALKERS) * (g - f) * g
        for i in range(NDIM):
            si = sdim(i)
            inv_s2 = 1.0 / (si * si)
            t = x[i] - mdim(i)
            gm = jnp.sum(coef * t * inv_s2, keepdims=True)
            gs = jnp.sum(coef * (t * t * inv_s2 / si - 1.0 / si),
                         keepdims=True)
            ms_s[0:1, i:i + 1] = mdim(i) - LR * gm
            ms_s[1:2, i:i + 1] = jnp.maximum(sdim(i) - LR * gs, EPS)

    @pl.when(k == NSTEP)
    def _emit():
        poscat = jnp.concatenate([pos_s[j] for j in range(D)], axis=1)
        out_ref[...] = lax.dot(poscat, perm_ref[...],
                               precision=lax.Precision.HIGHEST)


def _perm_matrix():
    a = jnp.arange(D * WC)
    j, c = a // WC, a % WC
    p = jnp.zeros((D * WC, D * WC), jnp.float32)
    return p.at[a, D * c + j].set(1.0)


def kernel(mean, sigma, init_pos, z_noise, u_noise):
    del init_pos
    z_t = z_noise.reshape(NSTEP + 1, NWALKERS, D).transpose(0, 2, 1)
    z_t = z_t.reshape(NSTEP + 1, D, WR, WC)
    u_r = u_noise.reshape(NSTEP, WR, WC)
    mean_r = mean.reshape(1, NDIM)
    sigma_r = sigma.reshape(1, NDIM)

    out = pl.pallas_call(
        _tc_body,
        grid=(NSTEP + 1,),
        in_specs=[
            pl.BlockSpec((1, NDIM), lambda k: (0, 0)),
            pl.BlockSpec((1, NDIM), lambda k: (0, 0)),
            pl.BlockSpec((D * WC, D * WC), lambda k: (0, 0)),
            pl.BlockSpec((1, D, WR, WC), lambda k: (k, 0, 0, 0)),
            pl.BlockSpec((1, WR, WC), lambda k: (jnp.maximum(k - 1, 0), 0, 0)),
        ],
        out_specs=pl.BlockSpec((WR, D * WC), lambda k: (0, 0)),
        out_shape=jax.ShapeDtypeStruct((WR, D * WC), jnp.float32),
        scratch_shapes=[
            pltpu.VMEM((2, 128), jnp.float32),
            pltpu.VMEM((D, WR, WC), jnp.float32),
        ],
    )(mean_r, sigma_r, _perm_matrix(), z_t, u_r)

    return out.reshape(NWALKERS, D)


# speedup vs baseline: 2.3474x; 2.3474x over previous
import math

import jax
import jax.numpy as jnp
from jax import lax
from jax.experimental import pallas as pl
from jax.experimental.pallas import tpu as pltpu

SQRT_2PI = math.sqrt(2.0 * math.pi)
NWALKERS = 65536
NELEC = 4
NDIM = 3
NSTEP = 16
RE = NWALKERS * NELEC
RR, RC = 2048, 128
LR = 0.2
EPS = 1e-7


def _body(mean_ref, sigma_ref, z_ref, u_ref, out_ref, ms_s):
    k = pl.program_id(0)

    @pl.when(k == 0)
    def _init():
        ms_s[0:1, :] = jnp.concatenate(
            [mean_ref[...], jnp.zeros((1, 128 - NDIM), jnp.float32)], axis=1)
        ms_s[1:2, :] = jnp.concatenate(
            [sigma_ref[...], jnp.zeros((1, 128 - NDIM), jnp.float32)], axis=1)

    def mdim(i):
        return ms_s[0:1, i:i + 1]

    def sdim(i):
        return ms_s[1:2, i:i + 1]

    x = [mdim(i) + sdim(i) * z_ref[i, 0] for i in range(NDIM)]

    @pl.when(k == 0)
    def _init_pos():
        for i in range(NDIM):
            out_ref[i] = x[i]

    @pl.when(k > 0)
    def _step():
        c4 = lax.broadcasted_iota(jnp.int32, (RR, RC), 1) % NELEC
        base = c4 == 0

        p = [jnp.exp(-(xi * xi) / 2.0) / SQRT_2PI for xi in x]
        p3 = (p[0] * p[1]) * p[2]
        f = p3
        for e in range(1, NELEC):
            f = f * jnp.roll(p3, -e, axis=1)

        g = None
        for i in range(NDIM):
            t = x[i] - mdim(i)
            gt = jnp.exp(-(t * t) / (2.0 * sdim(i) * sdim(i))) / (
                SQRT_2PI * sdim(i))
            g = gt if g is None else g * gt

        r = jnp.where(base, f / g, -jnp.inf)
        mmax = jnp.max(r)

        fm = jnp.where(base, f, 0.0)
        gm = jnp.where(base, g, 0.0)
        fb = fm
        gb = gm
        for e in range(1, NELEC):
            fb = fb + jnp.roll(fm, e, axis=1)
            gb = gb + jnp.roll(gm, e, axis=1)

        u = u_ref[0]
        accepted = (gb * u) * mmax < fb
        for i in range(NDIM):
            out_ref[i] = jnp.where(accepted, x[i], out_ref[i])

        coef = jnp.where(base, (2.0 / NWALKERS) * (g - f) * g, 0.0)
        s0 = jnp.sum(coef, keepdims=True)
        for i in range(NDIM):
            si = sdim(i)
            inv_s2 = 1.0 / (si * si)
            t = x[i] - mdim(i)
            ai = jnp.sum(coef * t, keepdims=True)
            bi = jnp.sum(coef * (t * t), keepdims=True)
            gm_i = ai * inv_s2
            gs_i = bi * inv_s2 / si - s0 / si
            ms_s[0:1, i:i + 1] = mdim(i) - LR * gm_i
            ms_s[1:2, i:i + 1] = jnp.maximum(sdim(i) - LR * gs_i, EPS)


def kernel(mean, sigma, init_pos, z_noise, u_noise):
    del init_pos
    zc = z_noise.transpose(2, 0, 1).reshape(NDIM, NSTEP + 1, RR, RC)
    u4 = jnp.repeat(u_noise, NELEC, axis=1).reshape(NSTEP, RR, RC)
    mean_r = mean.reshape(1, NDIM)
    sigma_r = sigma.reshape(1, NDIM)

    out = pl.pallas_call(
        _body,
        grid=(NSTEP + 1,),
        in_specs=[
            pl.BlockSpec((1, NDIM), lambda k: (0, 0)),
            pl.BlockSpec((1, NDIM), lambda k: (0, 0)),
            pl.BlockSpec((NDIM, 1, RR, RC), lambda k: (0, k, 0, 0)),
            pl.BlockSpec((1, RR, RC), lambda k: (jnp.maximum(k - 1, 0), 0, 0)),
        ],
        out_specs=pl.BlockSpec((NDIM, RR, RC), lambda k: (0, 0, 0)),
        out_shape=jax.ShapeDtypeStruct((NDIM, RR, RC), jnp.float32),
        scratch_shapes=[pltpu.VMEM((2, 128), jnp.float32)],
    )(mean_r, sigma_r, zc, u4)

    return out.reshape(NDIM, RE).transpose(1, 0).reshape(NWALKERS,
                                                         NELEC * NDIM)


# speedup vs baseline: 2.3475x; 1.0001x over previous
import math

import jax
import jax.numpy as jnp
from jax import lax
from jax.experimental import pallas as pl
from jax.experimental.pallas import tpu as pltpu

SQRT_2PI = math.sqrt(2.0 * math.pi)
NWALKERS = 65536
NELEC = 4
NDIM = 3
NSTEP = 16
RE = NWALKERS * NELEC
RR, RC = 2048, 128
LR = 0.2
EPS = 1e-7


def _body(mean_ref, sigma_ref, z_ref, u_ref, out_ref, ms_s):
    k = pl.program_id(0)

    @pl.when(k == 0)
    def _init():
        ms_s[0:1, :] = jnp.concatenate(
            [mean_ref[...], jnp.zeros((1, 128 - NDIM), jnp.float32)], axis=1)
        ms_s[1:2, :] = jnp.concatenate(
            [sigma_ref[...], jnp.zeros((1, 128 - NDIM), jnp.float32)], axis=1)

    def mdim(i):
        return ms_s[0:1, i:i + 1]

    def sdim(i):
        return ms_s[1:2, i:i + 1]

    x = [mdim(i) + sdim(i) * z_ref[i, 0] for i in range(NDIM)]

    @pl.when(k == 0)
    def _init_pos():
        for i in range(NDIM):
            out_ref[i] = x[i]

    @pl.when(k > 0)
    def _step():
        c4 = lax.broadcasted_iota(jnp.int32, (RR, RC), 1) % NELEC
        base = c4 == 0

        p = [jnp.exp(-(xi * xi) / 2.0) / SQRT_2PI for xi in x]
        p3 = (p[0] * p[1]) * p[2]
        f = p3
        for e in range(1, NELEC):
            f = f * jnp.roll(p3, -e, axis=1)

        g = None
        for i in range(NDIM):
            t = x[i] - mdim(i)
            gt = jnp.exp(-(t * t) / (2.0 * sdim(i) * sdim(i))) / (
                SQRT_2PI * sdim(i))
            g = gt if g is None else g * gt

        r = jnp.where(base, f / g, -jnp.inf)
        mmax = jnp.max(r)

        fm = jnp.where(base, f, 0.0)
        gm = jnp.where(base, g, 0.0)
        fb = fm
        gb = gm
        for e in range(1, NELEC):
            fb = fb + jnp.roll(fm, e, axis=1)
            gb = gb + jnp.roll(gm, e, axis=1)

        u = u_ref[0]
        accepted = (gb * u) * mmax < fb
        for i in range(NDIM):
            out_ref[i] = jnp.where(accepted, x[i], out_ref[i])

        coef = jnp.where(base, (2.0 / NWALKERS) * (g - f) * g, 0.0)
        s0 = jnp.sum(coef, keepdims=True)
        for i in range(NDIM):
            si = sdim(i)
            inv_s2 = 1.0 / (si * si)
            t = x[i] - mdim(i)
            ai = jnp.sum(coef * t, keepdims=True)
            bi = jnp.sum(coef * (t * t), keepdims=True)
            gm_i = ai * inv_s2
            gs_i = bi * inv_s2 / si - s0 / si
            ms_s[0:1, i:i + 1] = mdim(i) - LR * gm_i
            ms_s[1:2, i:i + 1] = jnp.maximum(sdim(i) - LR * gs_i, EPS)


def kernel(mean, sigma, init_pos, z_noise, u_noise):
    del init_pos
    zc = z_noise.transpose(2, 0, 1).reshape(NDIM, NSTEP + 1, RR, RC)
    u4 = jnp.broadcast_to(u_noise[:, :, None], (NSTEP, NWALKERS, NELEC))
    u4 = u4.reshape(NSTEP, RR, RC)
    mean_r = mean.reshape(1, NDIM)
    sigma_r = sigma.reshape(1, NDIM)

    out = pl.pallas_call(
        _body,
        grid=(NSTEP + 1,),
        in_specs=[
            pl.BlockSpec((1, NDIM), lambda k: (0, 0)),
            pl.BlockSpec((1, NDIM), lambda k: (0, 0)),
            pl.BlockSpec((NDIM, 1, RR, RC), lambda k: (0, k, 0, 0)),
            pl.BlockSpec((1, RR, RC), lambda k: (jnp.maximum(k - 1, 0), 0, 0)),
        ],
        out_specs=pl.BlockSpec((NDIM, RR, RC), lambda k: (0, 0, 0)),
        out_shape=jax.ShapeDtypeStruct((NDIM, RR, RC), jnp.float32),
        scratch_shapes=[pltpu.VMEM((2, 128), jnp.float32)],
    )(mean_r, sigma_r, zc, u4)

    return out.reshape(NDIM, RE).transpose(1, 0).reshape(NWALKERS,
                                                         NELEC * NDIM)
